# baseline (speedup 1.0000x reference)
import functools

import jax
import jax.numpy as jnp
from jax import lax
from jax.experimental import pallas as pl
from jax.experimental.pallas import tpu as pltpu
from jax.experimental.pallas import tpu_sc as plsc

NC = 2
NS = 16
NW = NC * NS
TM = 256


def _router_body(x_ref, semb_ref, rw_x_ref, rw_s_ref, sel_ref, wts_ref):
    x = x_ref[...]
    logits = lax.dot_general(x, rw_x_ref[...], (((1,), (0,)), ((), ())),
                             preferred_element_type=jnp.float32)
    logits += lax.dot_general(semb_ref[...], rw_s_ref[...],
                              (((1,), (0,)), ((), ())),
                              preferred_element_type=jnp.float32)
    probs = jax.nn.softmax(logits, axis=-1)
    e = probs.shape[-1]
    iota = lax.broadcasted_iota(jnp.int32, probs.shape, 1)
    big = jnp.int32(e + 1)
    m1 = jnp.max(probs, axis=-1, keepdims=True)
    am1 = jnp.min(jnp.where(probs == m1, iota, big), axis=-1, keepdims=True)
    probs2 = jnp.where(iota == am1, -jnp.inf, probs)
    m2 = jnp.max(probs2, axis=-1, keepdims=True)
    am2 = jnp.min(jnp.where(probs2 == m2, iota, big), axis=-1, keepdims=True)
    s = m1 + m2
    sel_ref[...] = jnp.concatenate([am1, am2], axis=1)
    wts_ref[...] = jnp.concatenate([m1 / s, m2 / s], axis=1)


def _router(x2, scale_emb, router_W, d, se, n_experts):
    s = x2.shape[0]
    tb = 256 if s % 256 == 0 else s
    return pl.pallas_call(
        _router_body,
        grid=(s // tb,),
        in_specs=[
            pl.BlockSpec((tb, d), lambda t: (t, 0)),
            pl.BlockSpec((1, se), lambda t: (0, 0)),
            pl.BlockSpec((d, n_experts), lambda t: (0, 0)),
            pl.BlockSpec((se, n_experts), lambda t: (0, 0)),
        ],
        out_specs=[
            pl.BlockSpec((tb, 2), lambda t: (t, 0)),
            pl.BlockSpec((tb, 2), lambda t: (t, 0)),
        ],
        out_shape=[
            jax.ShapeDtypeStruct((s, 2), jnp.int32),
            jax.ShapeDtypeStruct((s, 2), jnp.float32),
        ],
    )(x2, scale_emb, router_W[:d], router_W[d:])


def _sc_sort(sel_flat, n_experts, n_assign, ntot, nblocks):
    nv = n_assign // 16

    def body(sel_hbm, stok_hbm, pos_hbm, blk_hbm, selv, stokv, posv, blkv):
        wid = lax.axis_index("s") * NC + lax.axis_index("c")

        @pl.when(wid == 0)
        def _():
            pltpu.sync_copy(sel_hbm, selv)
            iota16 = lax.iota(jnp.int32, 16)
            zeros = jnp.zeros((16,), jnp.int32)

            def zinit(i, _):
                stokv[pl.ds(i * 16, 16)] = zeros
                return 0
            lax.fori_loop(0, ntot // 16, zinit, 0)

            def splat(v):
                return jnp.full((16,), v, jnp.int32)

            evecs = [jnp.full((16,), e, jnp.int32)
                     for e in range(n_experts)]

            def hist(i, cnts):
                a = selv[pl.ds(i * 16, 16)]
                return tuple(
                    cnts[e] + jnp.sum((a == evecs[e]).astype(jnp.int32))
                    for e in range(n_experts))
            cnts = lax.fori_loop(0, nv, hist,
                                 (jnp.int32(0),) * n_experts)

            start_blk = jnp.int32(0)
            starts = []
            ends_blk = []
            for e in range(n_experts):
                starts.append(start_blk * TM)
                pb = (cnts[e] + (TM - 1)) // TM
                start_blk = start_blk + pb
                ends_blk.append(start_blk)

            emax = jnp.full((16,), n_experts - 1, jnp.int32)
            for c in range((nblocks + 15) // 16):
                bvec = iota16 + jnp.full((16,), c * 16, jnp.int32)
                acc = zeros
                for e in range(n_experts):
                    acc = acc + (bvec >= splat(ends_blk[e])).astype(jnp.int32)
                blkv[pl.ds(c * 16, 16)] = jnp.minimum(acc, emax)

            ones = jnp.full((16,), 1, jnp.int32)

            def scat(i, offs):
                a = selv[pl.ds(i * 16, 16)]
                jvec = splat(i * 16) + iota16
                tok = lax.shift_right_logical(jvec, ones)
                pos_vec = zeros
                offs = list(offs)
                for e in range(n_experts):
                    m = a == evecs[e]
                    mi = m.astype(jnp.int32)
                    pref = plsc.cumsum(mi) - mi
                    slots = splat(offs[e]) + pref
                    plsc.store_scatter(stokv, [slots], tok, mask=m)
                    pos_vec = jnp.where(m, slots, pos_vec)
                    offs[e] = offs[e] + jnp.sum(mi)
                posv[pl.ds(i * 16, 16)] = pos_vec
                return tuple(offs)
            lax.fori_loop(0, nv, scat, tuple(starts))

            pltpu.sync_copy(stokv, stok_hbm)
            pltpu.sync_copy(posv, pos_hbm)
            pltpu.sync_copy(blkv, blk_hbm)

    nblk_pad = ((nblocks + 15) // 16) * 16
    f = functools.partial(
        pl.kernel, body,
        mesh=plsc.VectorSubcoreMesh(core_axis_name="c", subcore_axis_name="s"),
        compiler_params=pltpu.CompilerParams(needs_layout_passes=False),
        out_type=[
            jax.ShapeDtypeStruct((ntot,), jnp.int32),
            jax.ShapeDtypeStruct((n_assign,), jnp.int32),
            jax.ShapeDtypeStruct((nblk_pad,), jnp.int32),
        ],
        scratch_types=[
            pltpu.VMEM((n_assign,), jnp.int32),
            pltpu.VMEM((ntot,), jnp.int32),
            pltpu.VMEM((n_assign,), jnp.int32),
            pltpu.VMEM((nblk_pad,), jnp.int32),
        ],
    )()
    return f(sel_flat)


def _sc_gather(x_i32, stok, ntot, dwords):
    rows_per = ntot // NW

    def body(x_hbm, stok_hbm, out_hbm, idxv, rowsv, sem):
        wid = lax.axis_index("s") * NC + lax.axis_index("c")
        base = wid * rows_per
        pltpu.sync_copy(stok_hbm.at[pl.ds(base, rows_per)], idxv)
        pltpu.async_copy(x_hbm.at[idxv], rowsv, sem).wait()
        pltpu.sync_copy(rowsv, out_hbm.at[pl.ds(base, rows_per)])

    f = functools.partial(
        pl.kernel, body,
        mesh=plsc.VectorSubcoreMesh(core_axis_name="c", subcore_axis_name="s"),
        compiler_params=pltpu.CompilerParams(needs_layout_passes=False),
        out_type=jax.ShapeDtypeStruct((ntot, dwords), jnp.int32),
        scratch_types=[
            pltpu.VMEM((rows_per,), jnp.int32),
            pltpu.VMEM((rows_per, dwords), jnp.int32),
            pltpu.SemaphoreType.DMA,
        ],
    )()
    return f(x_i32, stok)


def _gffn_body(be_ref, xs_ref, w1_ref, b1_ref, w2_ref, b2_ref, y_ref):
    h = lax.dot_general(xs_ref[...], w1_ref[0], (((1,), (0,)), ((), ())),
                        preferred_element_type=jnp.float32)
    h += b1_ref[0].astype(jnp.float32)
    h = 0.5 * h * (1.0 + lax.erf(h * 0.7071067811865476))
    y = lax.dot_general(h.astype(jnp.bfloat16), w2_ref[0],
                        (((1,), (0,)), ((), ())),
                        preferred_element_type=jnp.float32)
    y_ref[...] = y + b2_ref[0].astype(jnp.float32)


def _grouped_ffn(blk_e, xs_bf, w1_bf, b1r, w2_bf, b2r, nblocks, d, hidden,
                 ntot):
    grid_spec = pltpu.PrefetchScalarGridSpec(
        num_scalar_prefetch=1,
        grid=(nblocks,),
        in_specs=[
            pl.BlockSpec((TM, d), lambda b, be: (b, 0)),
            pl.BlockSpec((1, d, hidden), lambda b, be: (be[b], 0, 0)),
            pl.BlockSpec((1, 1, hidden), lambda b, be: (be[b], 0, 0)),
            pl.BlockSpec((1, hidden, d), lambda b, be: (be[b], 0, 0)),
            pl.BlockSpec((1, 1, d), lambda b, be: (be[b], 0, 0)),
        ],
        out_specs=pl.BlockSpec((TM, d), lambda b, be: (b, 0)),
    )
    return pl.pallas_call(
        _gffn_body,
        grid_spec=grid_spec,
        out_shape=jax.ShapeDtypeStruct((ntot, d), jnp.float32),
    )(blk_e, xs_bf, w1_bf, b1r, w2_bf, b2r)


def _sc_combine(y_sorted, pos_flat, wts_flat, s, d):
    tok_per = s // NW
    tchunk = 16
    nchunk = tok_per // tchunk

    def body(y_hbm, pos_hbm, wts_hbm, out_hbm, posv, wv, rowsv, obuf, sem):
        wid = lax.axis_index("s") * NC + lax.axis_index("c")
        t0 = wid * tok_per
        for c in range(nchunk):
            tc0 = t0 + c * tchunk
            pltpu.sync_copy(pos_hbm.at[pl.ds(2 * tc0, 2 * tchunk)], posv)
            pltpu.sync_copy(wts_hbm.at[pl.ds(2 * tc0, 2 * tchunk)], wv)
            pltpu.async_copy(y_hbm.at[posv], rowsv, sem).wait()

            def per_tok(t, _):
                i0 = jnp.full((16,), 2 * t, jnp.int32)
                w0 = plsc.load_gather(wv, [i0])
                w1 = plsc.load_gather(wv, [i0 + 1])

                def per_grp(g, _):
                    slc = pl.ds(g * 16, 16)
                    obuf[t, slc] = (rowsv[2 * t, slc] * w0
                                    + rowsv[2 * t + 1, slc] * w1)
                    return 0
                lax.fori_loop(0, d // 16, per_grp, 0)
                return 0
            lax.fori_loop(0, tchunk, per_tok, 0)
            pltpu.sync_copy(obuf, out_hbm.at[pl.ds(tc0, tchunk)])

    f = functools.partial(
        pl.kernel, body,
        mesh=plsc.VectorSubcoreMesh(core_axis_name="c", subcore_axis_name="s"),
        compiler_params=pltpu.CompilerParams(needs_layout_passes=False),
        out_type=jax.ShapeDtypeStruct((s, d), jnp.float32),
        scratch_types=[
            pltpu.VMEM((2 * tchunk,), jnp.int32),
            pltpu.VMEM((2 * tchunk,), jnp.float32),
            pltpu.VMEM((2 * tchunk, d), jnp.float32),
            pltpu.VMEM((tchunk, d), jnp.float32),
            pltpu.SemaphoreType.DMA,
        ],
    )()
    return f(y_sorted, pos_flat, wts_flat)


def kernel(x, scale_idx, scale_embeddings, router_W, W1, b1, W2, b2):
    b, s, d = x.shape
    n_experts, _, hidden = W1.shape
    se = scale_embeddings.shape[-1]
    x2 = x.reshape(s, d)
    scale_emb = lax.dynamic_slice_in_dim(scale_embeddings, scale_idx, 1,
                                         axis=0)

    n_assign = 2 * s
    nblocks = n_assign // TM + n_experts
    ntot = nblocks * TM

    sel, wts = _router(x2, scale_emb, router_W, d, se, n_experts)

    stok, pos, blk_e = _sc_sort(sel.reshape(n_assign), n_experts, n_assign,
                                ntot, nblocks)
    x_bf = x2.astype(jnp.bfloat16)
    x_i32 = lax.bitcast_convert_type(
        x_bf.reshape(s, d // 2, 2), jnp.int32)
    xs_i32 = _sc_gather(x_i32, stok, ntot, d // 2)
    xs_bf = lax.bitcast_convert_type(xs_i32, jnp.bfloat16).reshape(ntot, d)

    w1_bf = W1.astype(jnp.bfloat16)
    w2_bf = W2.astype(jnp.bfloat16)
    y_sorted = _grouped_ffn(blk_e, xs_bf, w1_bf,
                            b1.reshape(n_experts, 1, hidden), w2_bf,
                            b2.reshape(n_experts, 1, d), nblocks, d, hidden,
                            ntot)

    out = _sc_combine(y_sorted, pos, wts.reshape(n_assign), s, d)
    return out.reshape(b, s, d)


# speedup vs baseline: 1.0262x; 1.0262x over previous
import functools

import jax
import jax.numpy as jnp
from jax import lax
from jax.experimental import pallas as pl
from jax.experimental.pallas import tpu as pltpu
from jax.experimental.pallas import tpu_sc as plsc

NC = 2
NS = 16
NW = NC * NS
TM = 256


def _router_body(x_ref, semb_ref, rw_x_ref, rw_s_ref, sel_ref, wts_ref):
    x = x_ref[...]
    logits = lax.dot_general(x, rw_x_ref[...], (((1,), (0,)), ((), ())),
                             preferred_element_type=jnp.float32)
    logits += lax.dot_general(semb_ref[...], rw_s_ref[...],
                              (((1,), (0,)), ((), ())),
                              preferred_element_type=jnp.float32)
    probs = jax.nn.softmax(logits, axis=-1)
    e = probs.shape[-1]
    iota = lax.broadcasted_iota(jnp.int32, probs.shape, 1)
    big = jnp.int32(e + 1)
    m1 = jnp.max(probs, axis=-1, keepdims=True)
    am1 = jnp.min(jnp.where(probs == m1, iota, big), axis=-1, keepdims=True)
    probs2 = jnp.where(iota == am1, -jnp.inf, probs)
    m2 = jnp.max(probs2, axis=-1, keepdims=True)
    am2 = jnp.min(jnp.where(probs2 == m2, iota, big), axis=-1, keepdims=True)
    s = m1 + m2
    sel_ref[...] = jnp.concatenate([am1, am2], axis=1)
    wts_ref[...] = jnp.concatenate([m1 / s, m2 / s], axis=1)


def _router(x2, scale_emb, router_W, d, se, n_experts):
    s = x2.shape[0]
    tb = 256 if s % 256 == 0 else s
    return pl.pallas_call(
        _router_body,
        grid=(s // tb,),
        in_specs=[
            pl.BlockSpec((tb, d), lambda t: (t, 0)),
            pl.BlockSpec((1, se), lambda t: (0, 0)),
            pl.BlockSpec((d, n_experts), lambda t: (0, 0)),
            pl.BlockSpec((se, n_experts), lambda t: (0, 0)),
        ],
        out_specs=[
            pl.BlockSpec((tb, 2), lambda t: (t, 0)),
            pl.BlockSpec((tb, 2), lambda t: (t, 0)),
        ],
        out_shape=[
            jax.ShapeDtypeStruct((s, 2), jnp.int32),
            jax.ShapeDtypeStruct((s, 2), jnp.float32),
        ],
    )(x2, scale_emb, router_W[:d], router_W[d:])


def _sc_sort(sel_flat, n_experts, n_assign, ntot, nblocks):
    nv = n_assign // 16

    def body(sel_hbm, stok_hbm, pos_hbm, blk_hbm, selv, stokv, posv, blkv):
        wid = lax.axis_index("s") * NC + lax.axis_index("c")

        @pl.when(wid == 0)
        def _():
            pltpu.sync_copy(sel_hbm, selv)
            iota16 = lax.iota(jnp.int32, 16)
            zeros = jnp.zeros((16,), jnp.int32)

            def zinit(i, _):
                stokv[pl.ds(i * 16, 16)] = zeros
                return 0
            lax.fori_loop(0, ntot // 16, zinit, 0)

            def splat(v):
                return jnp.full((16,), v, jnp.int32)

            evecs = [jnp.full((16,), e, jnp.int32)
                     for e in range(n_experts)]

            def hist(i, cnts):
                a = selv[pl.ds(i * 16, 16)]
                return tuple(
                    cnts[e] + jnp.sum((a == evecs[e]).astype(jnp.int32))
                    for e in range(n_experts))
            cnts = lax.fori_loop(0, nv, hist,
                                 (jnp.int32(0),) * n_experts)

            start_blk = jnp.int32(0)
            starts = []
            ends_blk = []
            for e in range(n_experts):
                starts.append(start_blk * TM)
                pb = (cnts[e] + (TM - 1)) // TM
                start_blk = start_blk + pb
                ends_blk.append(start_blk)

            emax = jnp.full((16,), n_experts - 1, jnp.int32)
            for c in range((nblocks + 15) // 16):
                bvec = iota16 + jnp.full((16,), c * 16, jnp.int32)
                acc = zeros
                for e in range(n_experts):
                    acc = acc + (bvec >= splat(ends_blk[e])).astype(jnp.int32)
                blkv[pl.ds(c * 16, 16)] = jnp.minimum(acc, emax)

            ones = jnp.full((16,), 1, jnp.int32)

            def scat(i, offs):
                a = selv[pl.ds(i * 16, 16)]
                jvec = splat(i * 16) + iota16
                tok = lax.shift_right_logical(jvec, ones)
                pos_vec = zeros
                offs = list(offs)
                for e in range(n_experts):
                    m = a == evecs[e]
                    mi = m.astype(jnp.int32)
                    pref = plsc.cumsum(mi) - mi
                    slots = splat(offs[e]) + pref
                    plsc.store_scatter(stokv, [slots], tok, mask=m)
                    pos_vec = jnp.where(m, slots, pos_vec)
                    offs[e] = offs[e] + jnp.sum(mi)
                posv[pl.ds(i * 16, 16)] = pos_vec
                return tuple(offs)
            lax.fori_loop(0, nv, scat, tuple(starts))

            pltpu.sync_copy(stokv, stok_hbm)
            pltpu.sync_copy(posv, pos_hbm)
            pltpu.sync_copy(blkv, blk_hbm)

    nblk_pad = ((nblocks + 15) // 16) * 16
    f = functools.partial(
        pl.kernel, body,
        mesh=plsc.VectorSubcoreMesh(core_axis_name="c", subcore_axis_name="s"),
        compiler_params=pltpu.CompilerParams(needs_layout_passes=False),
        out_type=[
            jax.ShapeDtypeStruct((ntot,), jnp.int32),
            jax.ShapeDtypeStruct((n_assign,), jnp.int32),
            jax.ShapeDtypeStruct((nblk_pad,), jnp.int32),
        ],
        scratch_types=[
            pltpu.VMEM((n_assign,), jnp.int32),
            pltpu.VMEM((ntot,), jnp.int32),
            pltpu.VMEM((n_assign,), jnp.int32),
            pltpu.VMEM((nblk_pad,), jnp.int32),
        ],
    )()
    return f(sel_flat)


def _sc_gather(x_i32, stok, ntot, dwords):
    rows_per = ntot // NW

    def body(x_hbm, stok_hbm, out_hbm, idxv, rowsv, sem):
        wid = lax.axis_index("s") * NC + lax.axis_index("c")
        base = wid * rows_per
        pltpu.sync_copy(stok_hbm.at[pl.ds(base, rows_per)], idxv)
        pltpu.async_copy(x_hbm.at[idxv], rowsv, sem).wait()
        pltpu.sync_copy(rowsv, out_hbm.at[pl.ds(base, rows_per)])

    f = functools.partial(
        pl.kernel, body,
        mesh=plsc.VectorSubcoreMesh(core_axis_name="c", subcore_axis_name="s"),
        compiler_params=pltpu.CompilerParams(needs_layout_passes=False),
        out_type=jax.ShapeDtypeStruct((ntot, dwords), jnp.int32),
        scratch_types=[
            pltpu.VMEM((rows_per,), jnp.int32),
            pltpu.VMEM((rows_per, dwords), jnp.int32),
            pltpu.SemaphoreType.DMA,
        ],
    )()
    return f(x_i32, stok)


def _gffn_body(be_ref, xs_ref, w1_ref, b1_ref, w2_ref, b2_ref, y_ref):
    h = lax.dot_general(xs_ref[...], w1_ref[0], (((1,), (0,)), ((), ())),
                        preferred_element_type=jnp.float32)
    h += b1_ref[0].astype(jnp.float32)
    h = 0.5 * h * (1.0 + lax.erf(h * 0.7071067811865476))
    y = lax.dot_general(h.astype(jnp.bfloat16), w2_ref[0],
                        (((1,), (0,)), ((), ())),
                        preferred_element_type=jnp.float32)
    y_ref[...] = y + b2_ref[0].astype(jnp.float32)


def _grouped_ffn(blk_e, xs_bf, w1_bf, b1r, w2_bf, b2r, nblocks, d, hidden,
                 ntot):
    grid_spec = pltpu.PrefetchScalarGridSpec(
        num_scalar_prefetch=1,
        grid=(nblocks,),
        in_specs=[
            pl.BlockSpec((TM, d), lambda b, be: (b, 0)),
            pl.BlockSpec((1, d, hidden), lambda b, be: (be[b], 0, 0)),
            pl.BlockSpec((1, 1, hidden), lambda b, be: (be[b], 0, 0)),
            pl.BlockSpec((1, hidden, d), lambda b, be: (be[b], 0, 0)),
            pl.BlockSpec((1, 1, d), lambda b, be: (be[b], 0, 0)),
        ],
        out_specs=pl.BlockSpec((TM, d), lambda b, be: (b, 0)),
    )
    return pl.pallas_call(
        _gffn_body,
        grid_spec=grid_spec,
        out_shape=jax.ShapeDtypeStruct((ntot, d), jnp.float32),
    )(blk_e, xs_bf, w1_bf, b1r, w2_bf, b2r)


def _sc_combine(y_sorted, pos_flat, wts_flat, s, d):
    tok_per = s // NW
    tchunk = 16
    nchunk = tok_per // tchunk

    def body(y_hbm, pos_hbm, wts_hbm, out_hbm, posv, wv, rowsv, obuf, sem):
        wid = lax.axis_index("s") * NC + lax.axis_index("c")
        t0 = wid * tok_per
        for c in range(nchunk):
            tc0 = t0 + c * tchunk
            pltpu.sync_copy(pos_hbm.at[pl.ds(2 * tc0, 2 * tchunk)], posv)
            pltpu.sync_copy(wts_hbm.at[pl.ds(2 * tc0, 2 * tchunk)], wv)
            pltpu.async_copy(y_hbm.at[posv], rowsv, sem).wait()

            ws = []
            for t in range(tchunk):
                i0 = jnp.full((16,), 2 * t, jnp.int32)
                ws.append((plsc.load_gather(wv, [i0]),
                           plsc.load_gather(wv, [i0 + 1])))

            def per_grp(g, _):
                slc = pl.ds(g * 16, 16)
                for t in range(tchunk):
                    obuf[t, slc] = (rowsv[2 * t, slc] * ws[t][0]
                                    + rowsv[2 * t + 1, slc] * ws[t][1])
                return 0
            lax.fori_loop(0, d // 16, per_grp, 0)
            pltpu.sync_copy(obuf, out_hbm.at[pl.ds(tc0, tchunk)])

    f = functools.partial(
        pl.kernel, body,
        mesh=plsc.VectorSubcoreMesh(core_axis_name="c", subcore_axis_name="s"),
        compiler_params=pltpu.CompilerParams(needs_layout_passes=False),
        out_type=jax.ShapeDtypeStruct((s, d), jnp.float32),
        scratch_types=[
            pltpu.VMEM((2 * tchunk,), jnp.int32),
            pltpu.VMEM((2 * tchunk,), jnp.float32),
            pltpu.VMEM((2 * tchunk, d), jnp.float32),
            pltpu.VMEM((tchunk, d), jnp.float32),
            pltpu.SemaphoreType.DMA,
        ],
    )()
    return f(y_sorted, pos_flat, wts_flat)


def kernel(x, scale_idx, scale_embeddings, router_W, W1, b1, W2, b2):
    b, s, d = x.shape
    n_experts, _, hidden = W1.shape
    se = scale_embeddings.shape[-1]
    x2 = x.reshape(s, d)
    scale_emb = lax.dynamic_slice_in_dim(scale_embeddings, scale_idx, 1,
                                         axis=0)

    n_assign = 2 * s
    nblocks = n_assign // TM + n_experts
    ntot = nblocks * TM

    sel, wts = _router(x2, scale_emb, router_W, d, se, n_experts)

    stok, pos, blk_e = _sc_sort(sel.reshape(n_assign), n_experts, n_assign,
                                ntot, nblocks)
    x_bf = x2.astype(jnp.bfloat16)
    x_i32 = lax.bitcast_convert_type(
        x_bf.reshape(s, d // 2, 2), jnp.int32)
    xs_i32 = _sc_gather(x_i32, stok, ntot, d // 2)
    xs_bf = lax.bitcast_convert_type(xs_i32, jnp.bfloat16).reshape(ntot, d)

    w1_bf = W1.astype(jnp.bfloat16)
    w2_bf = W2.astype(jnp.bfloat16)
    y_sorted = _grouped_ffn(blk_e, xs_bf, w1_bf,
                            b1.reshape(n_experts, 1, hidden), w2_bf,
                            b2.reshape(n_experts, 1, d), nblocks, d, hidden,
                            ntot)

    out = _sc_combine(y_sorted, pos, wts.reshape(n_assign), s, d)
    return out.reshape(b, s, d)


# speedup vs baseline: 1.1178x; 1.0893x over previous
import functools

import jax
import jax.numpy as jnp
from jax import lax
from jax.experimental import pallas as pl
from jax.experimental.pallas import tpu as pltpu
from jax.experimental.pallas import tpu_sc as plsc

NC = 2
NS = 16
NW = NC * NS
TM = 128


def _router_body(x_ref, semb_ref, rw_x_ref, rw_s_ref, sel_ref, wts_ref):
    x = x_ref[...]
    logits = lax.dot_general(x, rw_x_ref[...], (((1,), (0,)), ((), ())),
                             preferred_element_type=jnp.float32)
    logits += lax.dot_general(semb_ref[...], rw_s_ref[...],
                              (((1,), (0,)), ((), ())),
                              preferred_element_type=jnp.float32)
    probs = jax.nn.softmax(logits, axis=-1)
    e = probs.shape[-1]
    iota = lax.broadcasted_iota(jnp.int32, probs.shape, 1)
    big = jnp.int32(e + 1)
    m1 = jnp.max(probs, axis=-1, keepdims=True)
    am1 = jnp.min(jnp.where(probs == m1, iota, big), axis=-1, keepdims=True)
    probs2 = jnp.where(iota == am1, -jnp.inf, probs)
    m2 = jnp.max(probs2, axis=-1, keepdims=True)
    am2 = jnp.min(jnp.where(probs2 == m2, iota, big), axis=-1, keepdims=True)
    s = m1 + m2
    sel_ref[...] = jnp.concatenate([am1, am2], axis=1)
    wts_ref[...] = jnp.concatenate([m1 / s, m2 / s], axis=1)


def _router(x2, scale_emb, router_W, d, se, n_experts):
    s = x2.shape[0]
    tb = 256 if s % 256 == 0 else s
    return pl.pallas_call(
        _router_body,
        grid=(s // tb,),
        in_specs=[
            pl.BlockSpec((tb, d), lambda t: (t, 0)),
            pl.BlockSpec((1, se), lambda t: (0, 0)),
            pl.BlockSpec((d, n_experts), lambda t: (0, 0)),
            pl.BlockSpec((se, n_experts), lambda t: (0, 0)),
        ],
        out_specs=[
            pl.BlockSpec((tb, 2), lambda t: (t, 0)),
            pl.BlockSpec((tb, 2), lambda t: (t, 0)),
        ],
        out_shape=[
            jax.ShapeDtypeStruct((s, 2), jnp.int32),
            jax.ShapeDtypeStruct((s, 2), jnp.float32),
        ],
    )(x2, scale_emb, router_W[:d], router_W[d:])


def _sc_sort(sel_flat, n_experts, n_assign, ntot, nblocks):
    nv = n_assign // 16

    def body(sel_hbm, stok_hbm, pos_hbm, blk_hbm, selv, stokv, posv, blkv):
        wid = lax.axis_index("s") * NC + lax.axis_index("c")

        @pl.when(wid == 0)
        def _():
            pltpu.sync_copy(sel_hbm, selv)
            iota16 = lax.iota(jnp.int32, 16)
            zeros = jnp.zeros((16,), jnp.int32)

            def zinit(i, _):
                stokv[pl.ds(i * 16, 16)] = zeros
                return 0
            lax.fori_loop(0, ntot // 16, zinit, 0)

            def splat(v):
                return jnp.full((16,), v, jnp.int32)

            evecs = [jnp.full((16,), e, jnp.int32)
                     for e in range(n_experts)]

            def hist(i, cnts):
                a = selv[pl.ds(i * 16, 16)]
                return tuple(
                    cnts[e] + jnp.sum((a == evecs[e]).astype(jnp.int32))
                    for e in range(n_experts))
            cnts = lax.fori_loop(0, nv, hist,
                                 (jnp.int32(0),) * n_experts)

            start_blk = jnp.int32(0)
            starts = []
            ends_blk = []
            for e in range(n_experts):
                starts.append(start_blk * TM)
                pb = (cnts[e] + (TM - 1)) // TM
                start_blk = start_blk + pb
                ends_blk.append(start_blk)

            emax = jnp.full((16,), n_experts - 1, jnp.int32)
            for c in range((nblocks + 15) // 16):
                bvec = iota16 + jnp.full((16,), c * 16, jnp.int32)
                acc = zeros
                for e in range(n_experts):
                    acc = acc + (bvec >= splat(ends_blk[e])).astype(jnp.int32)
                blkv[pl.ds(c * 16, 16)] = jnp.minimum(acc, emax)

            ones = jnp.full((16,), 1, jnp.int32)

            def scat(i, offs):
                a = selv[pl.ds(i * 16, 16)]
                jvec = splat(i * 16) + iota16
                tok = lax.shift_right_logical(jvec, ones)
                pos_vec = zeros
                offs = list(offs)
                for e in range(n_experts):
                    m = a == evecs[e]
                    mi = m.astype(jnp.int32)
                    pref = plsc.cumsum(mi) - mi
                    slots = splat(offs[e]) + pref
                    plsc.store_scatter(stokv, [slots], tok, mask=m)
                    pos_vec = jnp.where(m, slots, pos_vec)
                    offs[e] = offs[e] + jnp.sum(mi)
                posv[pl.ds(i * 16, 16)] = pos_vec
                return tuple(offs)
            lax.fori_loop(0, nv, scat, tuple(starts))

            pltpu.sync_copy(stokv, stok_hbm)
            pltpu.sync_copy(posv, pos_hbm)
            pltpu.sync_copy(blkv, blk_hbm)

    nblk_pad = ((nblocks + 15) // 16) * 16
    f = functools.partial(
        pl.kernel, body,
        mesh=plsc.VectorSubcoreMesh(core_axis_name="c", subcore_axis_name="s"),
        compiler_params=pltpu.CompilerParams(needs_layout_passes=False),
        out_type=[
            jax.ShapeDtypeStruct((ntot,), jnp.int32),
            jax.ShapeDtypeStruct((n_assign,), jnp.int32),
            jax.ShapeDtypeStruct((nblk_pad,), jnp.int32),
        ],
        scratch_types=[
            pltpu.VMEM((n_assign,), jnp.int32),
            pltpu.VMEM((ntot,), jnp.int32),
            pltpu.VMEM((n_assign,), jnp.int32),
            pltpu.VMEM((nblk_pad,), jnp.int32),
        ],
    )()
    return f(sel_flat)


def _sc_gather(x_i32, stok, ntot, dwords):
    rows_per = ntot // NW

    def body(x_hbm, stok_hbm, out_hbm, idxv, rowsv, sem):
        wid = lax.axis_index("s") * NC + lax.axis_index("c")
        base = wid * rows_per
        pltpu.sync_copy(stok_hbm.at[pl.ds(base, rows_per)], idxv)
        pltpu.async_copy(x_hbm.at[idxv], rowsv, sem).wait()
        pltpu.sync_copy(rowsv, out_hbm.at[pl.ds(base, rows_per)])

    f = functools.partial(
        pl.kernel, body,
        mesh=plsc.VectorSubcoreMesh(core_axis_name="c", subcore_axis_name="s"),
        compiler_params=pltpu.CompilerParams(needs_layout_passes=False),
        out_type=jax.ShapeDtypeStruct((ntot, dwords), jnp.int32),
        scratch_types=[
            pltpu.VMEM((rows_per,), jnp.int32),
            pltpu.VMEM((rows_per, dwords), jnp.int32),
            pltpu.SemaphoreType.DMA,
        ],
    )()
    return f(x_i32, stok)


def _gffn_body(be_ref, xs_ref, w1_ref, b1_ref, w2_ref, b2_ref, y_ref):
    h = lax.dot_general(xs_ref[...], w1_ref[0], (((1,), (0,)), ((), ())),
                        preferred_element_type=jnp.float32)
    h += b1_ref[0].astype(jnp.float32)
    h = 0.5 * h * (1.0 + lax.erf(h * 0.7071067811865476))
    y = lax.dot_general(h.astype(jnp.bfloat16), w2_ref[0],
                        (((1,), (0,)), ((), ())),
                        preferred_element_type=jnp.float32)
    y_ref[...] = y + b2_ref[0].astype(jnp.float32)


def _grouped_ffn(blk_e, xs_bf, w1_bf, b1r, w2_bf, b2r, nblocks, d, hidden,
                 ntot):
    grid_spec = pltpu.PrefetchScalarGridSpec(
        num_scalar_prefetch=1,
        grid=(nblocks,),
        in_specs=[
            pl.BlockSpec((TM, d), lambda b, be: (b, 0)),
            pl.BlockSpec((1, d, hidden), lambda b, be: (be[b], 0, 0)),
            pl.BlockSpec((1, 1, hidden), lambda b, be: (be[b], 0, 0)),
            pl.BlockSpec((1, hidden, d), lambda b, be: (be[b], 0, 0)),
            pl.BlockSpec((1, 1, d), lambda b, be: (be[b], 0, 0)),
        ],
        out_specs=pl.BlockSpec((TM, d), lambda b, be: (b, 0)),
    )
    return pl.pallas_call(
        _gffn_body,
        grid_spec=grid_spec,
        out_shape=jax.ShapeDtypeStruct((ntot, d), jnp.float32),
    )(blk_e, xs_bf, w1_bf, b1r, w2_bf, b2r)


def _sc_combine(y_sorted, pos_flat, wts_flat, s, d):
    tok_per = s // NW
    tchunk = 16
    nchunk = tok_per // tchunk

    def body(y_hbm, pos_hbm, wts_hbm, out_hbm, posv, wv, rowsv, obuf, sem):
        wid = lax.axis_index("s") * NC + lax.axis_index("c")
        t0 = wid * tok_per
        for c in range(nchunk):
            tc0 = t0 + c * tchunk
            pltpu.sync_copy(pos_hbm.at[pl.ds(2 * tc0, 2 * tchunk)], posv)
            pltpu.sync_copy(wts_hbm.at[pl.ds(2 * tc0, 2 * tchunk)], wv)
            pltpu.async_copy(y_hbm.at[posv], rowsv, sem).wait()

            def per_tok(t, _):
                i0 = jnp.full((16,), 2 * t, jnp.int32)
                w0 = plsc.load_gather(wv, [i0])
                w1 = plsc.load_gather(wv, [i0 + 1])

                def per_grp(g, _):
                    slc = pl.ds(g * 16, 16)
                    obuf[t, slc] = (rowsv[2 * t, slc] * w0
                                    + rowsv[2 * t + 1, slc] * w1)
                    return 0
                lax.fori_loop(0, d // 16, per_grp, 0)
                return 0
            lax.fori_loop(0, tchunk, per_tok, 0)
            pltpu.sync_copy(obuf, out_hbm.at[pl.ds(tc0, tchunk)])

    f = functools.partial(
        pl.kernel, body,
        mesh=plsc.VectorSubcoreMesh(core_axis_name="c", subcore_axis_name="s"),
        compiler_params=pltpu.CompilerParams(needs_layout_passes=False),
        out_type=jax.ShapeDtypeStruct((s, d), jnp.float32),
        scratch_types=[
            pltpu.VMEM((2 * tchunk,), jnp.int32),
            pltpu.VMEM((2 * tchunk,), jnp.float32),
            pltpu.VMEM((2 * tchunk, d), jnp.float32),
            pltpu.VMEM((tchunk, d), jnp.float32),
            pltpu.SemaphoreType.DMA,
        ],
    )()
    return f(y_sorted, pos_flat, wts_flat)


def kernel(x, scale_idx, scale_embeddings, router_W, W1, b1, W2, b2):
    b, s, d = x.shape
    n_experts, _, hidden = W1.shape
    se = scale_embeddings.shape[-1]
    x2 = x.reshape(s, d)
    scale_emb = lax.dynamic_slice_in_dim(scale_embeddings, scale_idx, 1,
                                         axis=0)

    n_assign = 2 * s
    nblocks = n_assign // TM + n_experts
    ntot = nblocks * TM

    sel, wts = _router(x2, scale_emb, router_W, d, se, n_experts)

    stok, pos, blk_e = _sc_sort(sel.reshape(n_assign), n_experts, n_assign,
                                ntot, nblocks)
    x_bf = x2.astype(jnp.bfloat16)
    x_i32 = lax.bitcast_convert_type(
        x_bf.reshape(s, d // 2, 2), jnp.int32)
    xs_i32 = _sc_gather(x_i32, stok, ntot, d // 2)
    xs_bf = lax.bitcast_convert_type(xs_i32, jnp.bfloat16).reshape(ntot, d)

    w1_bf = W1.astype(jnp.bfloat16)
    w2_bf = W2.astype(jnp.bfloat16)
    y_sorted = _grouped_ffn(blk_e, xs_bf, w1_bf,
                            b1.reshape(n_experts, 1, hidden), w2_bf,
                            b2.reshape(n_experts, 1, d), nblocks, d, hidden,
                            ntot)

    out = _sc_combine(y_sorted, pos, wts.reshape(n_assign), s, d)
    return out.reshape(b, s, d)
